# baseline (speedup 1.0000x reference)
import functools

import jax
import jax.numpy as jnp
from jax import lax
from jax.experimental import pallas as pl
from jax.experimental.pallas import tpu as pltpu
from jax.experimental.pallas import tpu_sc as plsc

_NVOC = 1000000
_NUM_BINS = _NVOC + 1
_EMB_DIM = 32
_BATCH = 16384
_LANES = 16
_NBLK_FULL = _NVOC // 128
_BLK_PER_W = _NBLK_FULL // 16
_CB = 4
_CHUNK_COLS = _CB * 128
_NCHUNK = _BLK_PER_W // _CB
_XPIECE = 2048
_TRASH = _BATCH


def _hash_lanes(h, salt0, salt1):
    h = h * jnp.uint32(salt0) + jnp.uint32(salt1)
    h = h ^ (h >> jnp.uint32(16))
    h = h * jnp.uint32(0x45D9F3B)
    h = h ^ (h >> jnp.uint32(16))
    return (h % jnp.uint32(_NUM_BINS)).astype(jnp.int32)


def _iota16():
    return lax.broadcasted_iota(jnp.int32, (_LANES,), 0)


def _count(mask):
    return jnp.sum(mask.astype(jnp.int32))


def _gather_body(ww, salt0, salt1, x_hbm, tT_hbm, tail_hbm, ep_hbm,
                 xv, hit_i, hit_b, tmp_c, tmp_b, slab, tailslab,
                 stage, sem_slab0, sem_slab1, sem_st0, sem_st1):
    lo_blk = ww * _BLK_PER_W
    is_last = ww == 15
    hi_blk = jnp.where(is_last, _NBLK_FULL + 1, lo_blk + _BLK_PER_W)

    def piece_step(p, nh):
        pltpu.sync_copy(x_hbm.at[pl.ds(p * _XPIECE, _XPIECE)], xv)

        def lane_step(k, nh):
            xb = xv[pl.ds(k * _LANES, _LANES)].astype(jnp.uint32)
            h = _hash_lanes(xb, salt0, salt1)
            q = lax.shift_right_logical(h, 7)
            m = (q >= lo_blk) & (q < hi_blk)
            b = p * _XPIECE + k * _LANES + _iota16()
            cs = plsc.cumsum(m.astype(jnp.int32))
            pos = nh + cs - 1
            plsc.store_scatter(hit_i, [pos], h, mask=m)
            plsc.store_scatter(hit_b, [pos], b, mask=m)
            return nh + cs[_LANES - 1]

        return lax.fori_loop(0, _XPIECE // _LANES, lane_step, nh)

    nh = lax.fori_loop(0, _BATCH // _XPIECE, piece_step, 0)
    nhv = (nh + _LANES - 1) // _LANES

    sem_slab = (sem_slab0, sem_slab1)
    sem_st = (sem_st0, sem_st1)

    def fire_chunk(c, par):
        col0 = pl.multiple_of((lo_blk + c * _CB) * 128, 128)
        return pltpu.async_copy(
            tT_hbm.at[:, pl.ds(col0, _CHUNK_COLS)], slab.at[par],
            sem_slab[par])

    def wait_chunk(par):
        pltpu.make_async_copy(
            tT_hbm.at[:, pl.ds(0, _CHUNK_COLS)], slab.at[par], sem_slab[par]
        ).wait()

    def wait_stage(par):
        pltpu.make_async_copy(
            ep_hbm.at[pl.ds(0, _LANES)], stage.at[par], sem_st[par]).wait()

    def extract(src_ref, src_par, col_lo, col_hi, gg):
        def tmp_step(j, t):
            hv = hit_i[pl.ds(j * _LANES, _LANES)]
            bv = hit_b[pl.ds(j * _LANES, _LANES)]
            valid = (j * _LANES + _iota16()) < nh
            mc = valid & (hv >= col_lo) & (hv < col_hi)
            cs = plsc.cumsum(mc.astype(jnp.int32))
            pos = t + cs - 1
            plsc.store_scatter(tmp_c, [pos], hv - col_lo, mask=mc)
            plsc.store_scatter(tmp_b, [pos], bv, mask=mc)
            return t + cs[_LANES - 1]

        tn = lax.fori_loop(0, nhv, tmp_step, 0)

        def do_group(g, par):
            colv = tmp_c[pl.ds(g * _LANES, _LANES)]
            bv = tmp_b[pl.ds(g * _LANES, _LANES)]
            valid = (g * _LANES + _iota16()) < tn
            colv = jnp.where(valid, colv, 0)
            bs = jnp.where(valid, bv, _TRASH)
            for d in range(_EMB_DIM):
                v = plsc.load_gather(
                    src_ref, [jnp.full((_LANES,), src_par, jnp.int32),
                              jnp.full((_LANES,), d, jnp.int32), colv])
                plsc.store_scatter(
                    stage.at[par],
                    [_iota16(), jnp.full((_LANES,), d, jnp.int32)], v)
            pltpu.async_copy(stage.at[par], ep_hbm.at[bs], sem_st[par])

        def group_step(g, gg):
            par_t = gg % 2

            @pl.when(gg >= 2)
            def _():
                @pl.when(par_t == 0)
                def _():
                    wait_stage(0)

                @pl.when(par_t == 1)
                def _():
                    wait_stage(1)

            @pl.when(par_t == 0)
            def _():
                do_group(g, 0)

            @pl.when(par_t == 1)
            def _():
                do_group(g, 1)

            return gg + 1

        return lax.fori_loop(0, (tn + _LANES - 1) // _LANES, group_step, gg)

    fire_chunk(0, 0)
    fire_chunk(1, 1)

    def pair_step(p, gg):
        for par in range(2):
            c = 2 * p + par
            wait_chunk(par)
            col_lo = (lo_blk + c * _CB) * 128
            gg = extract(slab, par, col_lo, col_lo + _CHUNK_COLS, gg)

            @pl.when(c + 2 < _NCHUNK)
            def _():
                fire_chunk(c + 2, par)
        return gg

    gg = lax.fori_loop(0, _NCHUNK // 2, pair_step, 0)

    def _drain(gg):
        @pl.when(gg >= 1)
        def _():
            @pl.when((gg - 1) % 2 == 0)
            def _():
                wait_stage(0)

            @pl.when((gg - 1) % 2 == 1)
            def _():
                wait_stage(1)

        @pl.when(gg >= 2)
        def _():
            @pl.when(gg % 2 == 0)
            def _():
                wait_stage(0)

            @pl.when(gg % 2 == 1)
            def _():
                wait_stage(1)

    @pl.when(is_last)
    def _():
        ggl = gg
        pltpu.sync_copy(
            tT_hbm.at[:, pl.ds(_NBLK_FULL * 128 - _CHUNK_COLS, _CHUNK_COLS)],
            slab.at[0])
        ggl = extract(slab, 0, _NBLK_FULL * 128 - 4 * 128,
                      _NBLK_FULL * 128, ggl)
        pltpu.sync_copy(tail_hbm, tailslab.at[0])
        ggl = extract(tailslab, 0, _NBLK_FULL * 128, _NUM_BINS, ggl)
        _drain(ggl)

    @pl.when(jnp.logical_not(is_last))
    def _():
        _drain(gg)


def _ka_body(x_hbm, t1T_hbm, t2T_hbm, tail1_hbm, tail2_hbm,
             e1p_hbm, e2p_hbm,
             xv, hit_i, hit_b, tmp_c, tmp_b, slab, tailslab, stage,
             sem_slab0, sem_slab1, sem_st0, sem_st1):
    w = lax.axis_index("s") * 2 + lax.axis_index("c")
    ww = w % 16
    scratch = (xv, hit_i, hit_b, tmp_c, tmp_b, slab, tailslab, stage,
               sem_slab0, sem_slab1, sem_st0, sem_st1)

    @pl.when(w < 16)
    def _():
        _gather_body(ww, 6971, 7321, x_hbm, t1T_hbm, tail1_hbm, e1p_hbm,
                     *scratch)

    @pl.when(w >= 16)
    def _():
        _gather_body(ww, 7723, 7507, x_hbm, t2T_hbm, tail2_hbm, e2p_hbm,
                     *scratch)


def _kb_body(e1p_hbm, e2p_hbm, out_hbm, s1, s2, sem):
    w = lax.axis_index("s") * 2 + lax.axis_index("c")
    base = w * (_BATCH // 32)

    def chunk_step(c, _):
        row0 = base + c * 256
        pltpu.async_copy(e1p_hbm.at[pl.ds(row0, 256)], s1, sem)
        pltpu.async_copy(e2p_hbm.at[pl.ds(row0, 256)], s2, sem)
        pltpu.make_async_copy(e1p_hbm.at[pl.ds(0, 256)], s1, sem).wait()
        pltpu.make_async_copy(e2p_hbm.at[pl.ds(0, 256)], s2, sem).wait()

        def row_step(r, _):
            for h in range(_EMB_DIM // _LANES):
                sl = pl.ds(h * _LANES, _LANES)
                s1[r, sl] = s1[r, sl] * s2[r, sl]
            return 0
        lax.fori_loop(0, 256, row_step, 0)
        pltpu.sync_copy(s1, out_hbm.at[pl.ds(row0, 256)])
        return 0

    lax.fori_loop(0, _BATCH // 32 // 256, chunk_step, 0)


@jax.jit
def kernel(x, table1, table2):
    mesh = plsc.VectorSubcoreMesh(core_axis_name="c", subcore_axis_name="s")
    tail1 = jnp.pad(table1[_NBLK_FULL * 128:].T, ((0, 0), (0, 63)))
    tail2 = jnp.pad(table2[_NBLK_FULL * 128:].T, ((0, 0), (0, 63)))

    ka = pl.kernel(
        _ka_body,
        mesh=mesh,
        compiler_params=pltpu.CompilerParams(
            use_tc_tiling_on_sc=True, needs_layout_passes=False),
        out_type=(
            jax.ShapeDtypeStruct((_BATCH + 8, 128), jnp.float32),
            jax.ShapeDtypeStruct((_BATCH + 8, 128), jnp.float32),
        ),
        scratch_types=[
            pltpu.VMEM((_XPIECE,), jnp.int32),
            pltpu.VMEM((_BATCH + _LANES,), jnp.int32),
            pltpu.VMEM((_BATCH + _LANES,), jnp.int32),
            pltpu.VMEM((_BATCH + _LANES,), jnp.int32),
            pltpu.VMEM((_BATCH + _LANES,), jnp.int32),
            pltpu.VMEM((2, _EMB_DIM, _CHUNK_COLS), jnp.float32),
            pltpu.VMEM((1, _EMB_DIM, 128), jnp.float32),
            pltpu.VMEM((2, _LANES, 128), jnp.float32),
            pltpu.SemaphoreType.DMA,
            pltpu.SemaphoreType.DMA,
            pltpu.SemaphoreType.DMA,
            pltpu.SemaphoreType.DMA,
        ],
    )
    e1p, e2p = ka(x.astype(jnp.int32), table1.T, table2.T, tail1, tail2)

    kb = pl.kernel(
        _kb_body,
        mesh=mesh,
        compiler_params=pltpu.CompilerParams(use_tc_tiling_on_sc=True),
        out_type=jax.ShapeDtypeStruct((_BATCH, 128), jnp.float32),
        scratch_types=[
            pltpu.VMEM((256, 128), jnp.float32),
            pltpu.VMEM((256, 128), jnp.float32),
            pltpu.SemaphoreType.DMA,
        ],
    )
    outp = kb(e1p, e2p)
    return outp[:, :_EMB_DIM]
